# baseline (speedup 1.0000x reference)
import dataclasses
import functools
import math

import jax
import jax.numpy as jnp
from jax import lax
from jax.experimental import pallas as pl
from jax.experimental.pallas import tpu as pltpu
from jax.experimental.pallas import tpu_sc as plsc

_SIZE = 100000
_PAD = 0
_SMOOTHING = 0.1
_CONF = 1.0 - _SMOOTHING
_EPS = _SMOOTHING / (_SIZE - 2)
_C = (_SIZE - 2) * _EPS * math.log(_EPS) + _CONF * math.log(_CONF)

_N = 1024
_NWORK = 32
_RPW = _N // _NWORK
_PADW = 100096
_SEG = _PADW // 2
_CHUNK = 2048
_VALID0 = _SEG
_VALID1 = _SIZE - _SEG


def _sc_sum_and_gather(target, x):
    mesh = plsc.VectorSubcoreMesh(core_axis_name="c", subcore_axis_name="s")
    cp = pltpu.CompilerParams()
    if "needs_layout_passes" in pltpu.CompilerParams.__dataclass_fields__:
        cp = dataclasses.replace(cp, needs_layout_passes=False)
    nch = _SEG // _CHUNK
    tail = _SEG - nch * _CHUNK

    @functools.partial(
        pl.kernel,
        out_type=(
            jax.ShapeDtypeStruct((_N, 16), jnp.float32),
            jax.ShapeDtypeStruct((_N,), jnp.float32),
        ),
        mesh=mesh,
        compiler_params=cp,
        scratch_types=[
            pltpu.VMEM((_SEG,), jnp.float32),
            pltpu.VMEM((_SEG,), jnp.float32),
            pltpu.VMEM((_RPW, 16), jnp.float32),
            pltpu.VMEM((_RPW,), jnp.float32),
            pltpu.VMEM((32,), jnp.int32),
            pltpu.VMEM((16,), jnp.float32),
            pltpu.VMEM((16,), jnp.float32),
            pltpu.SemaphoreType.DMA,
            pltpu.SemaphoreType.DMA,
        ],
    )
    def sc_kernel(t_hbm, x_hbm, osum_hbm, og_hbm, buf0, buf1, osum_v,
                  og_v, idx_v, s0_v, g_v, sem0, sem1):
        wid = lax.axis_index("s") * 2 + lax.axis_index("c")
        base = wid * _RPW
        sems = (sem0, sem1)
        bufs = (buf0, buf1)
        lanes = lax.broadcasted_iota(jnp.int32, (16,), 0)

        pltpu.sync_copy(t_hbm.at[pl.ds(base, _RPW)], idx_v)

        def fire(j, seg, slot):
            row = x_hbm.at[base + j]
            for c in range(nch):
                pltpu.async_copy(
                    row.at[pl.ds(seg * _SEG + c * _CHUNK, _CHUNK)],
                    bufs[slot].at[pl.ds(c * _CHUNK, _CHUNK)],
                    sems[slot])
            toff = pl.multiple_of(
                jnp.asarray(seg * _SEG + nch * _CHUNK, jnp.int32)
                + (base + j) * 0, 128)
            pltpu.async_copy(
                row.at[pl.ds(toff, tail)],
                bufs[slot].at[pl.ds(nch * _CHUNK, tail)],
                sems[slot])

        def drain(slot):
            pltpu.make_async_copy(
                x_hbm.at[0].at[pl.ds(0, _SEG)], bufs[slot],
                sems[slot]).wait()

        def consume(j, seg, slot):
            row_buf = bufs[slot]
            valid = _VALID0 if seg == 0 else _VALID1
            n16 = valid // 16

            def add16(i, a):
                off = pl.multiple_of(i * 16, 16)
                return a + row_buf[pl.ds(off, 16)]

            acc = lax.fori_loop(0, n16, add16,
                                jnp.zeros((16,), jnp.float32), unroll=16)
            if seg == 0:
                s0_v[...] = acc
            else:
                osum_v.at[j][...] = s0_v[...] + acc

            goff = pl.multiple_of((j >> 4) << 4, 16)
            tv = idx_v[pl.ds(goff, 16)]
            rel = tv - seg * _SEG
            relc = jnp.minimum(
                jnp.maximum(rel, jnp.zeros_like(rel)),
                jnp.full_like(rel, valid - 1))
            gath = plsc.load_gather(row_buf, [relc])
            keep = ((rel >= 0) & (rel < valid)) & (lanes == (j & 15))
            g_v[...] = jnp.where(keep, gath, g_v[...])

        fire(0, 0, 0)
        fire(0, 1, 1)
        g_v[...] = jnp.zeros((16,), jnp.float32)

        @pl.loop(0, _RPW)
        def _(j):
            drain(0)
            consume(j, 0, 0)

            @pl.when(j + 1 < _RPW)
            def _():
                fire(j + 1, 0, 0)

            drain(1)
            consume(j, 1, 1)

            @pl.when(j + 1 < _RPW)
            def _():
                fire(j + 1, 1, 1)

            @pl.when((j & 15) == 15)
            def _():
                off = pl.multiple_of((j >> 4) << 4, 16)
                og_v[pl.ds(off, 16)] = g_v[...]
                g_v[...] = jnp.zeros((16,), jnp.float32)

        pltpu.sync_copy(osum_v, osum_hbm.at[pl.ds(base, _RPW)])
        pltpu.sync_copy(og_v, og_hbm.at[pl.ds(base, _RPW)])

    return sc_kernel(target, x)


def _combine_body(slab_ref, x_ref, g_ref, t_ref, out_ref):
    s = jnp.sum(slab_ref[...], axis=1, keepdims=True)
    x0 = x_ref[:, 0:1]
    g = g_ref[...]
    t = t_ref[...]
    row = _C - _EPS * s + _EPS * x0 + (_EPS - _CONF) * g
    row = jnp.where(t != _PAD, row, 0.0)
    out_ref[...] = jnp.sum(row, keepdims=True)


def _tc_combine(slab, x, g2, t2):
    out = pl.pallas_call(
        _combine_body,
        grid=(1,),
        in_specs=[
            pl.BlockSpec((_N, 16), lambda i: (0, 0)),
            pl.BlockSpec((_N, 128), lambda i: (0, 0)),
            pl.BlockSpec((_N, 1), lambda i: (0, 0)),
            pl.BlockSpec((_N, 1), lambda i: (0, 0)),
        ],
        out_specs=pl.BlockSpec((1, 1), lambda i: (0, 0)),
        out_shape=jax.ShapeDtypeStruct((1, 1), jnp.float32),
    )(slab, x, g2, t2)
    return out[0, 0]


def kernel(x, target):
    n, size = x.shape
    slab, g = _sc_sum_and_gather(target, x)
    return _tc_combine(slab, x, g.reshape(n, 1), target.reshape(n, 1))


# speedup vs baseline: 1.0023x; 1.0023x over previous
import dataclasses
import functools
import math

import jax
import jax.numpy as jnp
from jax import lax
from jax.experimental import pallas as pl
from jax.experimental.pallas import tpu as pltpu
from jax.experimental.pallas import tpu_sc as plsc

_SIZE = 100000
_PAD = 0
_SMOOTHING = 0.1
_CONF = 1.0 - _SMOOTHING
_EPS = _SMOOTHING / (_SIZE - 2)
_C = (_SIZE - 2) * _EPS * math.log(_EPS) + _CONF * math.log(_CONF)

_N = 1024
_NWORK = 32
_RPW = _N // _NWORK
_PADW = 100096
_SEG = _PADW // 2
_CHUNK = 2048
_VALID0 = _SEG
_VALID1 = _SIZE - _SEG


def _sc_sum_and_gather(target, x):
    mesh = plsc.VectorSubcoreMesh(core_axis_name="c", subcore_axis_name="s")
    cp = pltpu.CompilerParams()
    if "needs_layout_passes" in pltpu.CompilerParams.__dataclass_fields__:
        cp = dataclasses.replace(cp, needs_layout_passes=False)
    nch = _SEG // _CHUNK
    tail = _SEG - nch * _CHUNK

    @functools.partial(
        pl.kernel,
        out_type=(
            jax.ShapeDtypeStruct((_N, 16), jnp.float32),
            jax.ShapeDtypeStruct((_N,), jnp.float32),
        ),
        mesh=mesh,
        compiler_params=cp,
        scratch_types=[
            pltpu.VMEM((_SEG,), jnp.float32),
            pltpu.VMEM((_SEG,), jnp.float32),
            pltpu.VMEM((_RPW, 16), jnp.float32),
            pltpu.VMEM((_RPW,), jnp.float32),
            pltpu.VMEM((32,), jnp.int32),
            pltpu.VMEM((16,), jnp.float32),
            pltpu.VMEM((16,), jnp.float32),
            pltpu.SemaphoreType.DMA,
            pltpu.SemaphoreType.DMA,
        ],
    )
    def sc_kernel(t_hbm, x_hbm, osum_hbm, og_hbm, buf0, buf1, osum_v,
                  og_v, idx_v, s0_v, g_v, sem0, sem1):
        wid = lax.axis_index("s") * 2 + lax.axis_index("c")
        base = wid * _RPW
        sems = (sem0, sem1)
        bufs = (buf0, buf1)
        lanes = lax.broadcasted_iota(jnp.int32, (16,), 0)

        pltpu.sync_copy(t_hbm.at[pl.ds(base, _RPW)], idx_v)

        def fire(j, seg, slot):
            row = x_hbm.at[base + j]
            toff = pl.multiple_of(jnp.asarray(seg * _SEG, jnp.int32), 128)
            pltpu.async_copy(
                row.at[pl.ds(toff, _SEG)], bufs[slot], sems[slot])

        def drain(slot):
            pltpu.make_async_copy(
                x_hbm.at[0].at[pl.ds(0, _SEG)], bufs[slot],
                sems[slot]).wait()

        def consume(j, seg, slot):
            row_buf = bufs[slot]
            valid = _VALID0 if seg == 0 else _VALID1
            n16 = valid // 16

            def add16(i, a):
                off = pl.multiple_of(i * 16, 16)
                return a + row_buf[pl.ds(off, 16)]

            acc = lax.fori_loop(0, n16, add16,
                                jnp.zeros((16,), jnp.float32), unroll=16)
            if seg == 0:
                s0_v[...] = acc
            else:
                osum_v.at[j][...] = s0_v[...] + acc

            goff = pl.multiple_of((j >> 4) << 4, 16)
            tv = idx_v[pl.ds(goff, 16)]
            rel = tv - seg * _SEG
            relc = jnp.minimum(
                jnp.maximum(rel, jnp.zeros_like(rel)),
                jnp.full_like(rel, valid - 1))
            gath = plsc.load_gather(row_buf, [relc])
            keep = ((rel >= 0) & (rel < valid)) & (lanes == (j & 15))
            g_v[...] = jnp.where(keep, gath, g_v[...])

        fire(0, 0, 0)
        fire(0, 1, 1)
        g_v[...] = jnp.zeros((16,), jnp.float32)

        @pl.loop(0, _RPW)
        def _(j):
            drain(0)
            consume(j, 0, 0)

            @pl.when(j + 1 < _RPW)
            def _():
                fire(j + 1, 0, 0)

            drain(1)
            consume(j, 1, 1)

            @pl.when(j + 1 < _RPW)
            def _():
                fire(j + 1, 1, 1)

            @pl.when((j & 15) == 15)
            def _():
                off = pl.multiple_of((j >> 4) << 4, 16)
                og_v[pl.ds(off, 16)] = g_v[...]
                g_v[...] = jnp.zeros((16,), jnp.float32)

        pltpu.sync_copy(osum_v, osum_hbm.at[pl.ds(base, _RPW)])
        pltpu.sync_copy(og_v, og_hbm.at[pl.ds(base, _RPW)])

    return sc_kernel(target, x)


def _combine_body(slab_ref, x_ref, g_ref, t_ref, out_ref):
    s = jnp.sum(slab_ref[...], axis=1, keepdims=True)
    x0 = x_ref[:, 0:1]
    g = g_ref[...]
    t = t_ref[...]
    row = _C - _EPS * s + _EPS * x0 + (_EPS - _CONF) * g
    row = jnp.where(t != _PAD, row, 0.0)
    out_ref[...] = jnp.sum(row, keepdims=True)


def _tc_combine(slab, x, g2, t2):
    out = pl.pallas_call(
        _combine_body,
        grid=(1,),
        in_specs=[
            pl.BlockSpec((_N, 16), lambda i: (0, 0)),
            pl.BlockSpec((_N, 128), lambda i: (0, 0)),
            pl.BlockSpec((_N, 1), lambda i: (0, 0)),
            pl.BlockSpec((_N, 1), lambda i: (0, 0)),
        ],
        out_specs=pl.BlockSpec((1, 1), lambda i: (0, 0)),
        out_shape=jax.ShapeDtypeStruct((1, 1), jnp.float32),
    )(slab, x, g2, t2)
    return out[0, 0]


def kernel(x, target):
    n, size = x.shape
    slab, g = _sc_sum_and_gather(target, x)
    return _tc_combine(slab, x, g.reshape(n, 1), target.reshape(n, 1))


# speedup vs baseline: 1.2324x; 1.2296x over previous
import dataclasses
import functools
import math

import jax
import jax.numpy as jnp
from jax import lax
from jax.experimental import pallas as pl
from jax.experimental.pallas import tpu as pltpu
from jax.experimental.pallas import tpu_sc as plsc

_SIZE = 100000
_PAD = 0
_SMOOTHING = 0.1
_CONF = 1.0 - _SMOOTHING
_EPS = _SMOOTHING / (_SIZE - 2)
_C = (_SIZE - 2) * _EPS * math.log(_EPS) + _CONF * math.log(_CONF)

_N = 1024
_RB = 64
_NSTREAM = 4
_WAVE = 64


def _sc_gather(target, x):
    mesh = plsc.ScalarSubcoreMesh(axis_name="c", num_cores=2)
    half = _N // 2

    @functools.partial(
        pl.kernel,
        out_type=jax.ShapeDtypeStruct((_N,), jnp.float32),
        mesh=mesh,
        scratch_types=[
            pltpu.SMEM((half,), jnp.int32),
            pltpu.SMEM((_WAVE * 128,), jnp.float32),
            pltpu.SMEM((half,), jnp.float32),
            pltpu.SemaphoreType.DMA,
            pltpu.SemaphoreType.DMA,
        ],
    )
    def gather_kernel(t_hbm, x_hbm, out_hbm, idx_s, chunk_s, sel_s, sem,
                      gsem):
        cid = lax.axis_index("c")
        base = cid * half
        pltpu.async_copy(t_hbm.at[pl.ds(base, half)], idx_s, sem).wait()

        @pl.loop(0, half, step=_WAVE)
        def _(w):
            @pl.loop(0, _WAVE)
            def _(j):
                i = w + j
                t_al = pl.multiple_of((idx_s[i] >> 7) << 7, 128)
                pltpu.async_copy(
                    x_hbm.at[base + i].at[pl.ds(t_al, 128)],
                    chunk_s.at[pl.ds(j * 128, 128)], gsem)

            pltpu.make_async_copy(
                x_hbm.at[0].at[pl.ds(0, _WAVE * 128)], chunk_s, gsem).wait()

            @pl.loop(0, _WAVE)
            def _(j):
                i = w + j
                t = idx_s[i]
                sel_s[i] = chunk_s[j * 128 + (t & 127)]

        pltpu.async_copy(sel_s, out_hbm.at[pl.ds(base, half)], sem).wait()

    return gather_kernel(target, x)


def _rowsum_body(*refs):
    x_refs, (s_ref, x0_ref) = refs[:_NSTREAM], refs[_NSTREAM:]
    xs = [r[...] for r in x_refs]
    s_ref[...] = jnp.concatenate(
        [jnp.sum(xb, axis=1, keepdims=True) for xb in xs], axis=0)
    x0_ref[...] = jnp.concatenate([xb[:, 0:1] for xb in xs], axis=0)


def _tc_rowsum(x):
    stripe = _RB // _NSTREAM
    return pl.pallas_call(
        _rowsum_body,
        grid=(_N // _RB,),
        in_specs=[
            pl.BlockSpec((stripe, _SIZE),
                         lambda i, k=k: (i * _NSTREAM + k, 0))
            for k in range(_NSTREAM)
        ],
        out_specs=[
            pl.BlockSpec((_RB, 1), lambda i: (i, 0)),
            pl.BlockSpec((_RB, 1), lambda i: (i, 0)),
        ],
        out_shape=[
            jax.ShapeDtypeStruct((_N, 1), jnp.float32),
            jax.ShapeDtypeStruct((_N, 1), jnp.float32),
        ],
        compiler_params=pltpu.CompilerParams(
            dimension_semantics=("parallel",)),
    )(*([x] * _NSTREAM))


def _combine_body(s_ref, x0_ref, g_ref, t_ref, out_ref):
    s = s_ref[...]
    g = g_ref[...]
    x0 = x0_ref[...]
    t = t_ref[...]
    row = _C - _EPS * s + _EPS * x0 + (_EPS - _CONF) * g
    row = jnp.where(t != _PAD, row, 0.0)
    out_ref[...] = jnp.sum(row, keepdims=True)


def _tc_combine(s, x0, g2, t2):
    out = pl.pallas_call(
        _combine_body,
        out_shape=jax.ShapeDtypeStruct((1, 1), jnp.float32),
    )(s, x0, g2, t2)
    return out[0, 0]


def kernel(x, target):
    n, size = x.shape
    g = _sc_gather(target, x)
    s, x0 = _tc_rowsum(x)
    return _tc_combine(s, x0, g.reshape(n, 1), target.reshape(n, 1))


# speedup vs baseline: 1.2325x; 1.0000x over previous
import dataclasses
import functools
import math

import jax
import jax.numpy as jnp
from jax import lax
from jax.experimental import pallas as pl
from jax.experimental.pallas import tpu as pltpu
from jax.experimental.pallas import tpu_sc as plsc

_SIZE = 100000
_PAD = 0
_SMOOTHING = 0.1
_CONF = 1.0 - _SMOOTHING
_EPS = _SMOOTHING / (_SIZE - 2)
_C = (_SIZE - 2) * _EPS * math.log(_EPS) + _CONF * math.log(_CONF)

_N = 1024
_RB = 32
_NSTREAM = 4
_WAVE = 64


def _sc_gather(target, x):
    mesh = plsc.ScalarSubcoreMesh(axis_name="c", num_cores=2)
    half = _N // 2

    @functools.partial(
        pl.kernel,
        out_type=jax.ShapeDtypeStruct((_N,), jnp.float32),
        mesh=mesh,
        scratch_types=[
            pltpu.SMEM((half,), jnp.int32),
            pltpu.SMEM((_WAVE * 128,), jnp.float32),
            pltpu.SMEM((half,), jnp.float32),
            pltpu.SemaphoreType.DMA,
            pltpu.SemaphoreType.DMA,
        ],
    )
    def gather_kernel(t_hbm, x_hbm, out_hbm, idx_s, chunk_s, sel_s, sem,
                      gsem):
        cid = lax.axis_index("c")
        base = cid * half
        pltpu.async_copy(t_hbm.at[pl.ds(base, half)], idx_s, sem).wait()

        @pl.loop(0, half, step=_WAVE)
        def _(w):
            @pl.loop(0, _WAVE)
            def _(j):
                i = w + j
                t_al = pl.multiple_of((idx_s[i] >> 7) << 7, 128)
                pltpu.async_copy(
                    x_hbm.at[base + i].at[pl.ds(t_al, 128)],
                    chunk_s.at[pl.ds(j * 128, 128)], gsem)

            pltpu.make_async_copy(
                x_hbm.at[0].at[pl.ds(0, _WAVE * 128)], chunk_s, gsem).wait()

            @pl.loop(0, _WAVE)
            def _(j):
                i = w + j
                t = idx_s[i]
                sel_s[i] = chunk_s[j * 128 + (t & 127)]

        pltpu.async_copy(sel_s, out_hbm.at[pl.ds(base, half)], sem).wait()

    return gather_kernel(target, x)


def _rowsum_body(*refs):
    x_refs, (s_ref, x0_ref) = refs[:_NSTREAM], refs[_NSTREAM:]
    xs = [r[...] for r in x_refs]
    s_ref[...] = jnp.concatenate(
        [jnp.sum(xb, axis=1, keepdims=True) for xb in xs], axis=0)
    x0_ref[...] = jnp.concatenate([xb[:, 0:1] for xb in xs], axis=0)


def _tc_rowsum(x):
    stripe = _RB // _NSTREAM
    return pl.pallas_call(
        _rowsum_body,
        grid=(_N // _RB,),
        in_specs=[
            pl.BlockSpec((stripe, _SIZE),
                         lambda i, k=k: (i * _NSTREAM + k, 0))
            for k in range(_NSTREAM)
        ],
        out_specs=[
            pl.BlockSpec((_RB, 1), lambda i: (i, 0)),
            pl.BlockSpec((_RB, 1), lambda i: (i, 0)),
        ],
        out_shape=[
            jax.ShapeDtypeStruct((_N, 1), jnp.float32),
            jax.ShapeDtypeStruct((_N, 1), jnp.float32),
        ],
        compiler_params=pltpu.CompilerParams(
            dimension_semantics=("parallel",)),
    )(*([x] * _NSTREAM))


def _combine_body(s_ref, x0_ref, g_ref, t_ref, out_ref):
    s = s_ref[...]
    g = g_ref[...]
    x0 = x0_ref[...]
    t = t_ref[...]
    row = _C - _EPS * s + _EPS * x0 + (_EPS - _CONF) * g
    row = jnp.where(t != _PAD, row, 0.0)
    out_ref[...] = jnp.sum(row, keepdims=True)


def _tc_combine(s, x0, g2, t2):
    out = pl.pallas_call(
        _combine_body,
        out_shape=jax.ShapeDtypeStruct((1, 1), jnp.float32),
    )(s, x0, g2, t2)
    return out[0, 0]


def kernel(x, target):
    n, size = x.shape
    g = _sc_gather(target, x)
    s, x0 = _tc_rowsum(x)
    return _tc_combine(s, x0, g.reshape(n, 1), target.reshape(n, 1))
